# baseline (speedup 1.0000x reference)
import functools

import jax
import jax.numpy as jnp
from jax import lax
from jax.experimental import pallas as pl
from jax.experimental.pallas import tpu as pltpu
from jax.experimental.pallas import tpu_sc as plsc

N = 10000
NP = 10240
H = 128
HH = H // 2
L = 16
NC = 2
NS = 16
NW = NC * NS
CHUNK = 128
CHD = 80
CHA = 160
CHA2 = CHA // 2
EP = NW * CHD * CHUNK
ROWS_PER_TILE = NP // NS

_mesh = plsc.VectorSubcoreMesh(
    core_axis_name="c", subcore_axis_name="s", num_cores=NC, num_subcores=NS
)
_sc_params = pltpu.CompilerParams(
    needs_layout_passes=False, use_tc_tiling_on_sc=False
)


@functools.partial(
    pl.kernel,
    out_type=jax.ShapeDtypeStruct((NW, NP), jnp.float32),
    mesh=_mesh,
    scratch_types=[
        pltpu.VMEM((CHD, CHUNK), jnp.int32),
        pltpu.VMEM((CHD, CHUNK), jnp.float32),
        pltpu.VMEM((NP,), jnp.float32),
    ],
    compiler_params=_sc_params,
    name="sc_deg",
)
def _sc_deg(dst_hbm, ew_hbm, out_hbm, dst_v, ew_v, hist_v):
    c = lax.axis_index("c")
    s = lax.axis_index("s")
    w = c * NS + s

    pltpu.sync_copy(dst_hbm.at[w], dst_v)
    pltpu.sync_copy(ew_hbm.at[w], ew_v)

    zeros = jnp.zeros((L,), jnp.float32)

    def zero_body(i, _):
        hist_v[pl.ds(i * L, L)] = zeros
        return 0

    lax.fori_loop(0, NP // L, zero_body, 0)

    def edge_body(j, _):
        for g in range(CHUNK // L):
            idx = dst_v[j, pl.ds(g * L, L)]
            val = ew_v[j, pl.ds(g * L, L)]
            plsc.addupdate_scatter(hist_v, [idx], val)
        return 0

    lax.fori_loop(0, CHD, edge_body, 0)
    pltpu.sync_copy(hist_v, out_hbm.at[w])


@functools.partial(
    pl.kernel,
    out_type=jax.ShapeDtypeStruct((NC, NP, HH), jnp.float32),
    mesh=_mesh,
    scratch_types=[
        pltpu.VMEM((CHA, CHUNK), jnp.int32),
        pltpu.VMEM((CHA, CHUNK), jnp.float32),
        pltpu.VMEM((CHUNK,), jnp.int32),
        pltpu.VMEM((CHUNK,), jnp.int32),
        pltpu.VMEM((CHUNK,), jnp.int32),
        pltpu.VMEM((CHUNK,), jnp.int32),
        pltpu.VMEM((CHUNK, HH), jnp.float32),
        pltpu.VMEM((CHUNK, HH), jnp.float32),
        pltpu.VMEM((CHUNK, HH), jnp.float32),
        pltpu.VMEM((CHUNK, HH), jnp.float32),
        pltpu.VMEM_SHARED((NP, HH), jnp.float32),
        pltpu.SemaphoreType.DMA,
        pltpu.SemaphoreType.DMA,
        pltpu.SemaphoreType.DMA,
        pltpu.SemaphoreType.DMA,
    ],
    compiler_params=_sc_params,
    name="sc_acc",
)
def _sc_acc(y_hbm, sd_hbm, ew_hbm, out_hbm,
            sd_v, ew_v, gi0_v, gi1_v, si0_v, si1_v,
            g0_v, g1_v, s0_v, s1_v, acc_sh,
            gsem0, gsem1, ssem0, ssem1):
    c = lax.axis_index("c")
    s = lax.axis_index("s")
    y_c = y_hbm.at[c]

    pltpu.sync_copy(sd_hbm.at[s], sd_v)
    pltpu.sync_copy(ew_hbm.at[s], ew_v)

    def unpack_src(j, idx_ref):
        for g in range(CHUNK // L):
            p = sd_v[j, pl.ds(g * L, L)]
            idx_ref[pl.ds(g * L, L)] = p & jnp.int32(0xFFFF)

    def unpack_dst(j, idx_ref):
        for g in range(CHUNK // L):
            p = sd_v[j, pl.ds(g * L, L)]
            idx_ref[pl.ds(g * L, L)] = lax.shift_right_logical(p, 16)

    zeros = jnp.zeros((L,), jnp.float32)

    def zero_body(i, _):
        s0_v[i // (HH // L), pl.ds((i % (HH // L)) * L, L)] = zeros
        return 0

    lax.fori_loop(0, CHUNK * (HH // L), zero_body, 0)
    base = s * ROWS_PER_TILE

    def zero_acc(i, _):
        pltpu.sync_copy(s0_v, acc_sh.at[pl.ds(base + i * CHUNK, CHUNK)])
        return 0

    lax.fori_loop(0, ROWS_PER_TILE // CHUNK, zero_acc, 0)
    plsc.subcore_barrier()

    def scale(g_v, s_v, j):
        def scale_body(r):
            wv = plsc.load_gather(
                ew_v,
                [jnp.full((L,), j, jnp.int32), jnp.full((L,), r, jnp.int32)],
            )
            for cb in range(HH // L):
                s_v[r, pl.ds(cb * L, L)] = g_v[r, pl.ds(cb * L, L)] * wv

        plsc.parallel_loop(0, CHUNK, unroll=8)(scale_body)

    unpack_src(0, gi0_v)
    pltpu.async_copy(y_c.at[gi0_v], g0_v, gsem0)
    unpack_src(1, gi1_v)
    pltpu.async_copy(y_c.at[gi1_v], g1_v, gsem1)

    def pair_body(jj, _):
        j0 = 2 * jj
        pltpu.make_async_copy(y_c.at[gi0_v], g0_v, gsem0).wait()

        @pl.when(jj > 0)
        def _():
            pltpu.make_async_copy(s0_v, acc_sh.at[si0_v], ssem0).wait()

        scale(g0_v, s0_v, j0)
        unpack_dst(j0, si0_v)
        pltpu.async_copy(s0_v, acc_sh.at[si0_v], ssem0, add=True)
        unpack_src(jnp.minimum(j0 + 2, CHA - 1), gi0_v)
        pltpu.async_copy(y_c.at[gi0_v], g0_v, gsem0)

        pltpu.make_async_copy(y_c.at[gi1_v], g1_v, gsem1).wait()

        @pl.when(jj > 0)
        def _():
            pltpu.make_async_copy(s1_v, acc_sh.at[si1_v], ssem1).wait()

        scale(g1_v, s1_v, j0 + 1)
        unpack_dst(j0 + 1, si1_v)
        pltpu.async_copy(s1_v, acc_sh.at[si1_v], ssem1, add=True)
        unpack_src(jnp.minimum(j0 + 3, CHA - 1), gi1_v)
        pltpu.async_copy(y_c.at[gi1_v], g1_v, gsem1)
        return 0

    lax.fori_loop(0, CHA2, pair_body, 0)
    pltpu.make_async_copy(y_c.at[gi0_v], g0_v, gsem0).wait()
    pltpu.make_async_copy(y_c.at[gi1_v], g1_v, gsem1).wait()
    pltpu.make_async_copy(s0_v, acc_sh.at[si0_v], ssem0).wait()
    pltpu.make_async_copy(s1_v, acc_sh.at[si1_v], ssem1).wait()

    plsc.subcore_barrier()
    pltpu.sync_copy(acc_sh.at[pl.ds(base, ROWS_PER_TILE)],
                    out_hbm.at[c, pl.ds(base, ROWS_PER_TILE)])


BR = 256
GRID = NP // BR

_half = pl.BlockSpec((NC, BR, HH), lambda i: (0, i, 0))
_row = pl.BlockSpec((BR, H), lambda i: (i, 0))
_vec = pl.BlockSpec((BR,), lambda i: (i,))
_mat = pl.BlockSpec((H, H), lambda i: (0, 0))
_bias = pl.BlockSpec((1, H), lambda i: (0, 0))


def _split(y):
    return jnp.stack([y[:, :HH], y[:, HH:]], axis=0)


def _tc_first_body(x_ref, w_ref, dparts_ref, y_ref, dinv_ref):
    deg = jnp.sum(dparts_ref[...], axis=0) + 1.0
    d = lax.rsqrt(deg)
    xw = jnp.dot(x_ref[...], w_ref[...], preferred_element_type=jnp.float32)
    y_ref[...] = _split(d[:, None] * xw)
    dinv_ref[...] = d


_tc_first = pl.pallas_call(
    _tc_first_body,
    grid=(GRID,),
    in_specs=[_row, _mat, pl.BlockSpec((NW, BR), lambda i: (0, i))],
    out_specs=[_half, _vec],
    out_shape=[
        jax.ShapeDtypeStruct((NC, NP, HH), jnp.float32),
        jax.ShapeDtypeStruct((NP,), jnp.float32),
    ],
)


def _tc_layer_body(a_ref, y_ref, dinv_ref, b_ref, w_ref, out_ref, h_ref):
    d = dinv_ref[...]
    s = a_ref[...] + y_ref[...]
    full = jnp.concatenate([s[0], s[1]], axis=-1)
    h = jnp.maximum(d[:, None] * full + b_ref[...], 0.0)
    hw = jnp.dot(h, w_ref[...], preferred_element_type=jnp.float32)
    out_ref[...] = _split(d[:, None] * hw)
    h_ref[...] = h


_tc_layer = pl.pallas_call(
    _tc_layer_body,
    grid=(GRID,),
    in_specs=[_half, _half, _vec, _bias, _mat],
    out_specs=[_half, _row],
    out_shape=[
        jax.ShapeDtypeStruct((NC, NP, HH), jnp.float32),
        jax.ShapeDtypeStruct((NP, H), jnp.float32),
    ],
)


def _tc_head_body(h_ref, wfc_ref, bfc_ref, wout_ref, bout_ref, out_ref):
    f = jnp.dot(h_ref[...], wfc_ref[...], preferred_element_type=jnp.float32)
    f = jnp.maximum(f + bfc_ref[...], 0.0)
    o = jnp.dot(f, wout_ref[...], preferred_element_type=jnp.float32)
    out_ref[...] = o[:, 0] + bout_ref[0, 0]


_tc_head = pl.pallas_call(
    _tc_head_body,
    grid=(GRID,),
    in_specs=[_row, _mat, _bias,
              pl.BlockSpec((H, 1), lambda i: (0, 0)),
              pl.BlockSpec((1, 1), lambda i: (0, 0))],
    out_specs=pl.BlockSpec((BR,), lambda i: (i,)),
    out_shape=jax.ShapeDtypeStruct((NP,), jnp.float32),
)


def kernel(x, edge_index, edge_attr, W1, b1, W2, b2, W3, b3, Wfc, bfc, Wout, bout):
    E = edge_index.shape[1]
    pad_e = EP - E
    src_f = jnp.pad(edge_index[0], (0, pad_e))
    dst_f = jnp.pad(edge_index[1], (0, pad_e))
    ew_f = jnp.pad(edge_attr, (0, pad_e))
    dst_d = dst_f.reshape(NW, CHD, CHUNK)
    ew_d = ew_f.reshape(NW, CHD, CHUNK)
    sd_a = (src_f | (dst_f << 16)).reshape(NS, CHA, CHUNK)
    ew_a = ew_f.reshape(NS, CHA, CHUNK)
    xp = jnp.pad(x, ((0, NP - N), (0, 0)))

    deg_parts = _sc_deg(dst_d, ew_d)
    y1, dinv = _tc_first(xp, W1, deg_parts)

    bs = jnp.stack([b1, b2, b3]).reshape(3, 1, H)
    ws = jnp.stack([W2, W3, W2])

    def step(carry, params):
        y, _ = carry
        b, w = params
        acc = _sc_acc(y, sd_a, ew_a)
        y_next, h = _tc_layer(acc, y, dinv, b, w)
        return (y_next, h), None

    (_, h3), _ = lax.scan(step, (y1, jnp.zeros((NP, H), jnp.float32)), (bs, ws))
    out = _tc_head(h3, Wfc, bfc.reshape(1, H), Wout, bout.reshape(1, 1))
    return out[:N]


# speedup vs baseline: 1.0224x; 1.0224x over previous
import functools

import jax
import jax.numpy as jnp
from jax import lax
from jax.experimental import pallas as pl
from jax.experimental.pallas import tpu as pltpu
from jax.experimental.pallas import tpu_sc as plsc

N = 10000
NP = 10240
H = 128
HH = H // 2
L = 16
NC = 2
NS = 16
NW = NC * NS
CHUNK = 128
CHD = 80
CHA = 160
CHA2 = CHA // 2
EP = NW * CHD * CHUNK
ROWS_PER_TILE = NP // NS

_mesh = plsc.VectorSubcoreMesh(
    core_axis_name="c", subcore_axis_name="s", num_cores=NC, num_subcores=NS
)
_sc_params = pltpu.CompilerParams(
    needs_layout_passes=False, use_tc_tiling_on_sc=False
)


@functools.partial(
    pl.kernel,
    out_type=jax.ShapeDtypeStruct((NW, NP), jnp.float32),
    mesh=_mesh,
    scratch_types=[
        pltpu.VMEM((CHD, CHUNK), jnp.int32),
        pltpu.VMEM((CHD, CHUNK), jnp.float32),
        pltpu.VMEM((NP,), jnp.float32),
    ],
    compiler_params=_sc_params,
    name="sc_deg",
)
def _sc_deg(dst_hbm, ew_hbm, out_hbm, dst_v, ew_v, hist_v):
    c = lax.axis_index("c")
    s = lax.axis_index("s")
    w = c * NS + s

    pltpu.sync_copy(dst_hbm.at[w], dst_v)
    pltpu.sync_copy(ew_hbm.at[w], ew_v)

    zeros = jnp.zeros((L,), jnp.float32)

    def zero_body(i, _):
        hist_v[pl.ds(i * L, L)] = zeros
        return 0

    lax.fori_loop(0, NP // L, zero_body, 0)

    def edge_body(j, _):
        for g in range(CHUNK // L):
            idx = dst_v[j, pl.ds(g * L, L)]
            val = ew_v[j, pl.ds(g * L, L)]
            plsc.addupdate_scatter(hist_v, [idx], val)
        return 0

    lax.fori_loop(0, CHD, edge_body, 0)
    pltpu.sync_copy(hist_v, out_hbm.at[w])


@functools.partial(
    pl.kernel,
    out_type=jax.ShapeDtypeStruct((NC, NP, HH), jnp.float32),
    mesh=_mesh,
    scratch_types=[
        pltpu.VMEM((CHA, CHUNK), jnp.int32),
        pltpu.VMEM((CHA, CHUNK), jnp.float32),
        pltpu.VMEM((CHUNK,), jnp.int32),
        pltpu.VMEM((CHUNK,), jnp.int32),
        pltpu.VMEM((CHUNK,), jnp.int32),
        pltpu.VMEM((CHUNK,), jnp.int32),
        pltpu.VMEM((CHUNK, HH), jnp.float32),
        pltpu.VMEM((CHUNK, HH), jnp.float32),
        pltpu.VMEM((CHUNK, HH), jnp.float32),
        pltpu.VMEM((CHUNK, HH), jnp.float32),
        pltpu.VMEM_SHARED((NP, HH), jnp.float32),
        pltpu.SemaphoreType.DMA,
        pltpu.SemaphoreType.DMA,
        pltpu.SemaphoreType.DMA,
        pltpu.SemaphoreType.DMA,
    ],
    compiler_params=_sc_params,
    name="sc_acc",
)
def _sc_acc(y_hbm, sd_hbm, ew_hbm, out_hbm,
            sd_v, ew_v, gi0_v, gi1_v, si0_v, si1_v,
            g0_v, g1_v, s0_v, s1_v, acc_sh,
            gsem0, gsem1, ssem0, ssem1):
    c = lax.axis_index("c")
    s = lax.axis_index("s")
    y_c = y_hbm.at[c]

    pltpu.sync_copy(sd_hbm.at[s], sd_v)
    pltpu.sync_copy(ew_hbm.at[s], ew_v)

    def unpack_src(j, idx_ref):
        for g in range(CHUNK // L):
            p = sd_v[j, pl.ds(g * L, L)]
            idx_ref[pl.ds(g * L, L)] = p & jnp.int32(0xFFFF)

    def unpack_dst(j, idx_ref):
        for g in range(CHUNK // L):
            p = sd_v[j, pl.ds(g * L, L)]
            idx_ref[pl.ds(g * L, L)] = lax.shift_right_logical(p, 16)

    zeros = jnp.zeros((L,), jnp.float32)

    def zero_body(i, _):
        s0_v[i // (HH // L), pl.ds((i % (HH // L)) * L, L)] = zeros
        return 0

    lax.fori_loop(0, CHUNK * (HH // L), zero_body, 0)
    base = s * ROWS_PER_TILE

    def zero_acc(i, _):
        pltpu.sync_copy(s0_v, acc_sh.at[pl.ds(base + i * CHUNK, CHUNK)])
        return 0

    lax.fori_loop(0, ROWS_PER_TILE // CHUNK, zero_acc, 0)
    plsc.subcore_barrier()

    def scale(g_v, s_v, j):
        def scale_body(r):
            wv = plsc.load_gather(
                ew_v,
                [jnp.full((L,), j, jnp.int32), jnp.full((L,), r, jnp.int32)],
            )
            for cb in range(HH // L):
                s_v[r, pl.ds(cb * L, L)] = g_v[r, pl.ds(cb * L, L)] * wv

        plsc.parallel_loop(0, CHUNK, unroll=8)(scale_body)

    unpack_src(0, gi0_v)
    pltpu.async_copy(y_c.at[gi0_v], g0_v, gsem0)
    unpack_src(1, gi1_v)
    pltpu.async_copy(y_c.at[gi1_v], g1_v, gsem1)

    def pair_body(jj, _):
        j0 = 2 * jj
        pltpu.make_async_copy(y_c.at[gi0_v], g0_v, gsem0).wait()

        @pl.when(jj > 0)
        def _():
            pltpu.make_async_copy(s0_v, acc_sh.at[si0_v], ssem0).wait()

        scale(g0_v, s0_v, j0)
        unpack_dst(j0, si0_v)
        pltpu.async_copy(s0_v, acc_sh.at[si0_v], ssem0, add=True)
        unpack_src(jnp.minimum(j0 + 2, CHA - 1), gi0_v)
        pltpu.async_copy(y_c.at[gi0_v], g0_v, gsem0)

        pltpu.make_async_copy(y_c.at[gi1_v], g1_v, gsem1).wait()

        @pl.when(jj > 0)
        def _():
            pltpu.make_async_copy(s1_v, acc_sh.at[si1_v], ssem1).wait()

        scale(g1_v, s1_v, j0 + 1)
        unpack_dst(j0 + 1, si1_v)
        pltpu.async_copy(s1_v, acc_sh.at[si1_v], ssem1, add=True)
        unpack_src(jnp.minimum(j0 + 3, CHA - 1), gi1_v)
        pltpu.async_copy(y_c.at[gi1_v], g1_v, gsem1)
        return 0

    lax.fori_loop(0, CHA2, pair_body, 0)
    pltpu.make_async_copy(y_c.at[gi0_v], g0_v, gsem0).wait()
    pltpu.make_async_copy(y_c.at[gi1_v], g1_v, gsem1).wait()
    pltpu.make_async_copy(s0_v, acc_sh.at[si0_v], ssem0).wait()
    pltpu.make_async_copy(s1_v, acc_sh.at[si1_v], ssem1).wait()

    plsc.subcore_barrier()
    pltpu.sync_copy(acc_sh.at[pl.ds(base, ROWS_PER_TILE)],
                    out_hbm.at[c, pl.ds(base, ROWS_PER_TILE)])


BR = 256
GRID = NP // BR

_half = pl.BlockSpec((NC, BR, HH), lambda i: (0, i, 0))
_row = pl.BlockSpec((BR, H), lambda i: (i, 0))
_vec = pl.BlockSpec((BR,), lambda i: (i,))
_mat = pl.BlockSpec((H, H), lambda i: (0, 0))
_bias = pl.BlockSpec((1, H), lambda i: (0, 0))


def _split(y):
    return jnp.stack([y[:, :HH], y[:, HH:]], axis=0)


def _tc_first_body(x_ref, w_ref, dparts_ref, y_ref, dinv_ref):
    deg = jnp.sum(dparts_ref[...], axis=0) + 1.0
    d = lax.rsqrt(deg)
    xw = jnp.dot(x_ref[...], w_ref[...], preferred_element_type=jnp.float32)
    y_ref[...] = _split(d[:, None] * xw)
    dinv_ref[...] = d


_tc_first = pl.pallas_call(
    _tc_first_body,
    grid=(GRID,),
    in_specs=[_row, _mat, pl.BlockSpec((NW, BR), lambda i: (0, i))],
    out_specs=[_half, _vec],
    out_shape=[
        jax.ShapeDtypeStruct((NC, NP, HH), jnp.float32),
        jax.ShapeDtypeStruct((NP,), jnp.float32),
    ],
)


def _tc_layer_body(a_ref, y_ref, dinv_ref, b_ref, w_ref, out_ref, h_ref):
    d = dinv_ref[...]
    s = a_ref[...] + y_ref[...]
    full = jnp.concatenate([s[0], s[1]], axis=-1)
    h = jnp.maximum(d[:, None] * full + b_ref[...], 0.0)
    hw = jnp.dot(h, w_ref[...], preferred_element_type=jnp.float32)
    out_ref[...] = _split(d[:, None] * hw)
    h_ref[...] = h


_tc_layer = pl.pallas_call(
    _tc_layer_body,
    grid=(GRID,),
    in_specs=[_half, _half, _vec, _bias, _mat],
    out_specs=[_half, _row],
    out_shape=[
        jax.ShapeDtypeStruct((NC, NP, HH), jnp.float32),
        jax.ShapeDtypeStruct((NP, H), jnp.float32),
    ],
)


def _tc_head_body(h_ref, wfc_ref, bfc_ref, wout_ref, bout_ref, out_ref):
    f = jnp.dot(h_ref[...], wfc_ref[...], preferred_element_type=jnp.float32)
    f = jnp.maximum(f + bfc_ref[...], 0.0)
    o = jnp.dot(f, wout_ref[...], preferred_element_type=jnp.float32)
    out_ref[...] = o[:, 0] + bout_ref[0, 0]


_tc_head = pl.pallas_call(
    _tc_head_body,
    grid=(GRID,),
    in_specs=[_row, _mat, _bias,
              pl.BlockSpec((H, 1), lambda i: (0, 0)),
              pl.BlockSpec((1, 1), lambda i: (0, 0))],
    out_specs=pl.BlockSpec((BR,), lambda i: (i,)),
    out_shape=jax.ShapeDtypeStruct((NP,), jnp.float32),
)


def kernel(x, edge_index, edge_attr, W1, b1, W2, b2, W3, b3, Wfc, bfc, Wout, bout):
    E = edge_index.shape[1]
    pad_e = EP - E
    src_f = jnp.pad(edge_index[0], (0, pad_e))
    dst_f = jnp.pad(edge_index[1], (0, pad_e))
    ew_f = jnp.pad(edge_attr, (0, pad_e))
    dst_d = dst_f.reshape(NW, CHD, CHUNK)
    ew_d = ew_f.reshape(NW, CHD, CHUNK)
    sd_a = (src_f | (dst_f << 16)).reshape(NS, CHA, CHUNK)
    ew_a = ew_f.reshape(NS, CHA, CHUNK)
    xp = jnp.pad(x, ((0, NP - N), (0, 0)))

    deg_parts = _sc_deg(dst_d, ew_d)
    y1, dinv = _tc_first(xp, W1, deg_parts)

    acc1 = _sc_acc(y1, sd_a, ew_a)
    y2, _ = _tc_layer(acc1, y1, dinv, b1.reshape(1, H), W2)
    acc2 = _sc_acc(y2, sd_a, ew_a)
    y3, _ = _tc_layer(acc2, y2, dinv, b2.reshape(1, H), W3)
    acc3 = _sc_acc(y3, sd_a, ew_a)
    _, h3 = _tc_layer(acc3, y3, dinv, b3.reshape(1, H), W3)
    out = _tc_head(h3, Wfc, bfc.reshape(1, H), Wout, bout.reshape(1, 1))
    return out[:N]


# speedup vs baseline: 1.1391x; 1.1141x over previous
import functools

import jax
import jax.numpy as jnp
from jax import lax
from jax.experimental import pallas as pl
from jax.experimental.pallas import tpu as pltpu
from jax.experimental.pallas import tpu_sc as plsc

N = 10000
NP = 10240
H = 128
HH = H // 2
L = 16
NC = 2
NS = 16
NW = NC * NS
CHUNK = 128
CHD = 80
CHA = 160
CHA2 = CHA // 2
EP = NW * CHD * CHUNK
ROWS_PER_TILE = NP // NS

_mesh = plsc.VectorSubcoreMesh(
    core_axis_name="c", subcore_axis_name="s", num_cores=NC, num_subcores=NS
)
_sc_params = pltpu.CompilerParams(
    needs_layout_passes=False, use_tc_tiling_on_sc=False
)


@functools.partial(
    pl.kernel,
    out_type=jax.ShapeDtypeStruct((NC, NP), jnp.float32),
    mesh=_mesh,
    scratch_types=[
        pltpu.VMEM((CHD, CHUNK), jnp.int32),
        pltpu.VMEM((CHD, CHUNK), jnp.float32),
        pltpu.VMEM((NP,), jnp.float32),
        pltpu.VMEM((NS, ROWS_PER_TILE), jnp.float32),
        pltpu.VMEM((ROWS_PER_TILE,), jnp.float32),
        pltpu.VMEM_SHARED((NS, NP), jnp.float32),
    ],
    compiler_params=_sc_params,
    name="sc_deg",
)
def _sc_deg(dst_hbm, ew_hbm, out_hbm, dst_v, ew_v, hist_v, red_v, sum_v, stage_sh):
    c = lax.axis_index("c")
    s = lax.axis_index("s")
    w = c * NS + s

    pltpu.sync_copy(dst_hbm.at[w], dst_v)
    pltpu.sync_copy(ew_hbm.at[w], ew_v)

    zeros = jnp.zeros((L,), jnp.float32)

    def zero_body(i, _):
        hist_v[pl.ds(i * L, L)] = zeros
        return 0

    lax.fori_loop(0, NP // L, zero_body, 0)

    def edge_body(j, _):
        for g in range(CHUNK // L):
            idx = dst_v[j, pl.ds(g * L, L)]
            val = ew_v[j, pl.ds(g * L, L)]
            plsc.addupdate_scatter(hist_v, [idx], val)
        return 0

    lax.fori_loop(0, CHD, edge_body, 0)

    pltpu.sync_copy(hist_v, stage_sh.at[s])
    plsc.subcore_barrier()
    base = s * ROWS_PER_TILE
    pltpu.sync_copy(stage_sh.at[:, pl.ds(base, ROWS_PER_TILE)], red_v)

    def red_body(b, _):
        acc = red_v[0, pl.ds(b * L, L)]
        for t in range(1, NS):
            acc = acc + red_v[t, pl.ds(b * L, L)]
        sum_v[pl.ds(b * L, L)] = acc
        return 0

    lax.fori_loop(0, ROWS_PER_TILE // L, red_body, 0)
    pltpu.sync_copy(sum_v, out_hbm.at[c, pl.ds(base, ROWS_PER_TILE)])


@functools.partial(
    pl.kernel,
    out_type=jax.ShapeDtypeStruct((NC, NP, HH), jnp.float32),
    mesh=_mesh,
    scratch_types=[
        pltpu.VMEM((CHA, CHUNK), jnp.int32),
        pltpu.VMEM((CHA, CHUNK), jnp.int32),
        pltpu.VMEM((CHA, CHUNK), jnp.float32),
        pltpu.VMEM((CHUNK, HH), jnp.float32),
        pltpu.VMEM((CHUNK, HH), jnp.float32),
        pltpu.VMEM_SHARED((NP, HH), jnp.float32),
        pltpu.SemaphoreType.DMA,
        pltpu.SemaphoreType.DMA,
    ],
    compiler_params=_sc_params,
    name="sc_acc",
)
def _sc_acc(y_hbm, src_hbm, dst_hbm, ew_hbm, out_hbm,
            src_v, dst_v, ew_v, rows0_v, rows1_v, acc_sh, sem0, sem1):
    c = lax.axis_index("c")
    s = lax.axis_index("s")
    y_c = y_hbm.at[c]

    pltpu.sync_copy(src_hbm.at[s], src_v)
    pltpu.sync_copy(dst_hbm.at[s], dst_v)
    pltpu.sync_copy(ew_hbm.at[s], ew_v)

    zeros = jnp.zeros((L,), jnp.float32)

    def zero_body(i, _):
        rows0_v[i // (HH // L), pl.ds((i % (HH // L)) * L, L)] = zeros
        return 0

    lax.fori_loop(0, CHUNK * (HH // L), zero_body, 0)
    base = s * ROWS_PER_TILE
    for i in range(ROWS_PER_TILE // CHUNK):
        pltpu.sync_copy(rows0_v, acc_sh.at[pl.ds(base + i * CHUNK, CHUNK)])
    plsc.subcore_barrier()

    def scale(rows_v, j):
        def scale_body(r):
            wv = plsc.load_gather(
                ew_v,
                [jnp.full((L,), j, jnp.int32), jnp.full((L,), r, jnp.int32)],
            )
            for cb in range(HH // L):
                rows_v[r, pl.ds(cb * L, L)] = rows_v[r, pl.ds(cb * L, L)] * wv

        plsc.parallel_loop(0, CHUNK, unroll=8)(scale_body)

    pltpu.async_copy(y_c.at[src_v.at[0]], rows0_v, sem0)

    def pair_body(jj, _):
        j0 = 2 * jj
        pltpu.async_copy(y_c.at[src_v.at[j0 + 1]], rows1_v, sem1)
        pltpu.make_async_copy(y_c.at[src_v.at[j0]], rows0_v, sem0).wait()
        scale(rows0_v, j0)
        pltpu.sync_copy(rows0_v, acc_sh.at[dst_v.at[j0]], add=True)

        jn = jnp.minimum(j0 + 2, CHA - 1)
        pltpu.async_copy(y_c.at[src_v.at[jn]], rows0_v, sem0)
        pltpu.make_async_copy(y_c.at[src_v.at[j0 + 1]], rows1_v, sem1).wait()
        scale(rows1_v, j0 + 1)
        pltpu.sync_copy(rows1_v, acc_sh.at[dst_v.at[j0 + 1]], add=True)
        return 0

    lax.fori_loop(0, CHA2, pair_body, 0)
    pltpu.make_async_copy(y_c.at[src_v.at[CHA - 1]], rows0_v, sem0).wait()

    plsc.subcore_barrier()
    pltpu.sync_copy(acc_sh.at[pl.ds(base, ROWS_PER_TILE)],
                    out_hbm.at[c, pl.ds(base, ROWS_PER_TILE)])


BR = 256
GRID = NP // BR

_half = pl.BlockSpec((NC, BR, HH), lambda i: (0, i, 0))
_row = pl.BlockSpec((BR, H), lambda i: (i, 0))
_vec = pl.BlockSpec((BR,), lambda i: (i,))
_mat = pl.BlockSpec((H, H), lambda i: (0, 0))
_bias = pl.BlockSpec((1, H), lambda i: (0, 0))


def _split(y):
    return jnp.stack([y[:, :HH], y[:, HH:]], axis=0)


def _tc_first_body(x_ref, w_ref, d0_ref, d1_ref, y_ref, dinv_ref):
    d = lax.rsqrt(d0_ref[...] + d1_ref[...] + 1.0)
    xw = jnp.dot(x_ref[...], w_ref[...], preferred_element_type=jnp.float32)
    y_ref[...] = _split(d[:, None] * xw)
    dinv_ref[...] = d


_tc_first = pl.pallas_call(
    _tc_first_body,
    grid=(GRID,),
    in_specs=[_row, _mat, _vec, _vec],
    out_specs=[_half, _vec],
    out_shape=[
        jax.ShapeDtypeStruct((NC, NP, HH), jnp.float32),
        jax.ShapeDtypeStruct((NP,), jnp.float32),
    ],
)


def _tc_layer_body(a_ref, y_ref, dinv_ref, b_ref, w_ref, out_ref):
    d = dinv_ref[...]
    s = a_ref[...] + y_ref[...]
    full = jnp.concatenate([s[0], s[1]], axis=-1)
    h = jnp.maximum(d[:, None] * full + b_ref[...], 0.0)
    hw = jnp.dot(h, w_ref[...], preferred_element_type=jnp.float32)
    out_ref[...] = _split(d[:, None] * hw)


_tc_layer = pl.pallas_call(
    _tc_layer_body,
    grid=(GRID,),
    in_specs=[_half, _half, _vec, _bias, _mat],
    out_specs=_half,
    out_shape=jax.ShapeDtypeStruct((NC, NP, HH), jnp.float32),
)


def _tc_head_body(a_ref, y_ref, dinv_ref, b3_ref, wfc_ref, bfc_ref,
                  wout_ref, bout_ref, out_ref):
    d = dinv_ref[...]
    s = a_ref[...] + y_ref[...]
    full = jnp.concatenate([s[0], s[1]], axis=-1)
    h = jnp.maximum(d[:, None] * full + b3_ref[...], 0.0)
    f = jnp.dot(h, wfc_ref[...], preferred_element_type=jnp.float32)
    f = jnp.maximum(f + bfc_ref[...], 0.0)
    o = jnp.dot(f, wout_ref[...], preferred_element_type=jnp.float32)
    out_ref[...] = o[:, 0] + bout_ref[0, 0]


_tc_head = pl.pallas_call(
    _tc_head_body,
    grid=(GRID,),
    in_specs=[_half, _half, _vec, _bias, _mat, _bias,
              pl.BlockSpec((H, 1), lambda i: (0, 0)),
              pl.BlockSpec((1, 1), lambda i: (0, 0))],
    out_specs=pl.BlockSpec((BR,), lambda i: (i,)),
    out_shape=jax.ShapeDtypeStruct((NP,), jnp.float32),
)


def kernel(x, edge_index, edge_attr, W1, b1, W2, b2, W3, b3, Wfc, bfc, Wout, bout):
    E = edge_index.shape[1]
    pad_e = EP - E
    src_f = jnp.pad(edge_index[0], (0, pad_e))
    dst_f = jnp.pad(edge_index[1], (0, pad_e))
    ew_f = jnp.pad(edge_attr, (0, pad_e))
    dst_d = dst_f.reshape(NW, CHD, CHUNK)
    ew_d = ew_f.reshape(NW, CHD, CHUNK)
    src_a = src_f.reshape(NS, CHA, CHUNK)
    dst_a = dst_f.reshape(NS, CHA, CHUNK)
    ew_a = ew_f.reshape(NS, CHA, CHUNK)
    xp = jnp.pad(x, ((0, NP - N), (0, 0)))

    deg_parts = _sc_deg(dst_d, ew_d)
    y1, dinv = _tc_first(xp, W1, deg_parts[0], deg_parts[1])

    acc1 = _sc_acc(y1, src_a, dst_a, ew_a)
    y2 = _tc_layer(acc1, y1, dinv, b1.reshape(1, H), W2)

    acc2 = _sc_acc(y2, src_a, dst_a, ew_a)
    y3 = _tc_layer(acc2, y2, dinv, b2.reshape(1, H), W3)

    acc3 = _sc_acc(y3, src_a, dst_a, ew_a)
    out = _tc_head(acc3, y3, dinv, b3.reshape(1, H),
                   Wfc, bfc.reshape(1, H), Wout, bout.reshape(1, 1))
    return out[:N]
